# baseline (speedup 1.0000x reference)
import functools

import jax
import jax.numpy as jnp
from jax.experimental import pallas as pl
from jax.experimental.pallas import tpu as pltpu

_N = 2048
_RB = 256


def _rowmax_kernel(sim_ref, score_ref, idx_ref):
    x = (sim_ref[...] + 1.0) * 0.5
    m = jnp.max(x, axis=-1, keepdims=True)
    iota = jax.lax.broadcasted_iota(jnp.int32, x.shape, 2)
    first = jnp.min(jnp.where(x == m, iota, _N), axis=-1)
    score_ref[...] = m.reshape(1, 1, -1)
    idx_ref[...] = first.reshape(1, 1, -1)


@functools.partial(jax.jit, static_argnums=())
def kernel(importance, similarity, compressed_map):
    del importance, compressed_map
    b, r, n = similarity.shape
    rg = r // _RB
    grid = (b, rg)
    score, idx = pl.pallas_call(
        _rowmax_kernel,
        grid=grid,
        in_specs=[pl.BlockSpec((1, _RB, n), lambda i, j: (i, j, 0))],
        out_specs=[
            pl.BlockSpec((1, 1, _RB), lambda i, j: (i * rg + j, 0, 0)),
            pl.BlockSpec((1, 1, _RB), lambda i, j: (i * rg + j, 0, 0)),
        ],
        out_shape=[
            jax.ShapeDtypeStruct((b * rg, 1, _RB), jnp.float32),
            jax.ShapeDtypeStruct((b * rg, 1, _RB), jnp.int32),
        ],
        compiler_params=pltpu.CompilerParams(
            dimension_semantics=("parallel", "parallel"),
        ),
    )(similarity)
    return score.reshape(b, r), idx.reshape(b, r)


# speedup vs baseline: 1.2872x; 1.2872x over previous
import functools

import jax
import jax.numpy as jnp
from jax.experimental import pallas as pl
from jax.experimental.pallas import tpu as pltpu

_N = 2048
_RB = 512


def _rowmax_kernel(sim_ref, score_ref, idx_ref):
    x = sim_ref[...] * 0.5 + 0.5
    m = jnp.max(x, axis=-1, keepdims=True)
    iota = jax.lax.broadcasted_iota(jnp.int32, x.shape, 2)
    first = jnp.min(jnp.where(x == m, iota, _N), axis=-1)
    score_ref[...] = m.reshape(1, 1, -1)
    idx_ref[...] = first.reshape(1, 1, -1)


@functools.partial(jax.jit, static_argnums=())
def kernel(importance, similarity, compressed_map):
    del importance, compressed_map
    b, r, n = similarity.shape
    rg = r // _RB
    grid = (b, rg)
    score, idx = pl.pallas_call(
        _rowmax_kernel,
        grid=grid,
        in_specs=[pl.BlockSpec((1, _RB, n), lambda i, j: (i, j, 0))],
        out_specs=[
            pl.BlockSpec((1, 1, _RB), lambda i, j: (i * rg + j, 0, 0)),
            pl.BlockSpec((1, 1, _RB), lambda i, j: (i * rg + j, 0, 0)),
        ],
        out_shape=[
            jax.ShapeDtypeStruct((b * rg, 1, _RB), jnp.float32),
            jax.ShapeDtypeStruct((b * rg, 1, _RB), jnp.int32),
        ],
        compiler_params=pltpu.CompilerParams(
            dimension_semantics=("parallel", "parallel"),
        ),
    )(similarity)
    return score.reshape(b, r), idx.reshape(b, r)


# speedup vs baseline: 1.4766x; 1.1471x over previous
import functools

import jax
import jax.numpy as jnp
from jax.experimental import pallas as pl
from jax.experimental.pallas import tpu as pltpu

_N = 2048
_RB = 1024


def _rowmax_kernel(sim_ref, score_ref, idx_ref):
    x = sim_ref[...] * 0.5 + 0.5
    m = jnp.max(x, axis=-1, keepdims=True)
    iota = jax.lax.broadcasted_iota(jnp.int32, x.shape, 2)
    first = jnp.min(jnp.where(x == m, iota, _N), axis=-1)
    score_ref[...] = m.reshape(1, 1, -1)
    idx_ref[...] = first.reshape(1, 1, -1)


@functools.partial(jax.jit, static_argnums=())
def kernel(importance, similarity, compressed_map):
    del importance, compressed_map
    b, r, n = similarity.shape
    rg = r // _RB
    grid = (b, rg)
    score, idx = pl.pallas_call(
        _rowmax_kernel,
        grid=grid,
        in_specs=[pl.BlockSpec((1, _RB, n), lambda i, j: (i, j, 0))],
        out_specs=[
            pl.BlockSpec((1, 1, _RB), lambda i, j: (i * rg + j, 0, 0)),
            pl.BlockSpec((1, 1, _RB), lambda i, j: (i * rg + j, 0, 0)),
        ],
        out_shape=[
            jax.ShapeDtypeStruct((b * rg, 1, _RB), jnp.float32),
            jax.ShapeDtypeStruct((b * rg, 1, _RB), jnp.int32),
        ],
        compiler_params=pltpu.CompilerParams(
            dimension_semantics=("parallel", "parallel"),
        ),
    )(similarity)
    return score.reshape(b, r), idx.reshape(b, r)


# speedup vs baseline: 1.5956x; 1.0806x over previous
import functools

import jax
import jax.numpy as jnp
from jax.experimental import pallas as pl
from jax.experimental.pallas import tpu as pltpu

_N = 2048
_RB = 2048


def _rowmax_kernel(sim_ref, score_ref, idx_ref):
    x = sim_ref[...] * 0.5 + 0.5
    m = jnp.max(x, axis=-1, keepdims=True)
    iota = jax.lax.broadcasted_iota(jnp.int32, x.shape, 2)
    first = jnp.min(jnp.where(x == m, iota, _N), axis=-1)
    score_ref[...] = m.reshape(1, 1, -1)
    idx_ref[...] = first.reshape(1, 1, -1)


@functools.partial(jax.jit, static_argnums=())
def kernel(importance, similarity, compressed_map):
    del importance, compressed_map
    b, r, n = similarity.shape
    rg = r // _RB
    grid = (b, rg)
    score, idx = pl.pallas_call(
        _rowmax_kernel,
        grid=grid,
        in_specs=[pl.BlockSpec((1, _RB, n), lambda i, j: (i, j, 0))],
        out_specs=[
            pl.BlockSpec((1, 1, _RB), lambda i, j: (i * rg + j, 0, 0)),
            pl.BlockSpec((1, 1, _RB), lambda i, j: (i * rg + j, 0, 0)),
        ],
        out_shape=[
            jax.ShapeDtypeStruct((b * rg, 1, _RB), jnp.float32),
            jax.ShapeDtypeStruct((b * rg, 1, _RB), jnp.int32),
        ],
        compiler_params=pltpu.CompilerParams(
            dimension_semantics=("parallel", "parallel"),
        ),
    )(similarity)
    return score.reshape(b, r), idx.reshape(b, r)
